# baseline (speedup 1.0000x reference)
import functools

import jax
import jax.numpy as jnp
from jax import lax
from jax.experimental import pallas as pl
from jax.experimental.pallas import tpu as pltpu
from jax.experimental.pallas import tpu_sc as plsc

Q = 1024
D = 64
K_DB = 100000
NUM_CLASSES = 1000
TOP_K = 10

K_BLK = 4096
NKB = -(-K_DB // K_BLK)
K_PAD = K_BLK * NKB

OUT_PAD = 1024
INT_MAX = 2**31 - 1
NEG_INF = float("-inf")


Q_BLK = 1024
NQB = Q // Q_BLK
CAND_W = 32


def _topk_body(ypred_ref, feats_ref, out_ref,
               sim_ref, cv_ref, ci_ref, nv_ref, ni_ref):
    k = pl.program_id(1)

    @pl.when(k == 0)
    def _init():
        cv_ref[...] = jnp.full((Q_BLK, CAND_W), NEG_INF, jnp.float32)
        ci_ref[...] = jnp.full((Q_BLK, CAND_W), INT_MAX, jnp.int32)

    gcol = lax.broadcasted_iota(jnp.int32, (Q_BLK, K_BLK), 1) + k * K_BLK
    sim = lax.dot_general(
        ypred_ref[...], feats_ref[...],
        (((1,), (1,)), ((), ())),
        preferred_element_type=jnp.float32,
    )

    @pl.when(k == NKB - 1)
    def _store_masked():
        sim_ref[...] = jnp.where(gcol < K_DB, sim, NEG_INF)

    @pl.when(k < NKB - 1)
    def _store_plain():
        sim_ref[...] = sim

    lanec = lax.broadcasted_iota(jnp.int32, (Q_BLK, CAND_W), 1)
    lane16 = lax.broadcasted_iota(jnp.int32, (Q_BLK, 16), 1)

    thresh = cv_ref[:, 9:10]

    def pass_cond(state):
        t, m = state
        return jnp.logical_and(t < TOP_K // 2, jnp.any(m > thresh))

    def pass_body(state):
        t, m = state
        s = sim_ref[...]
        sel0 = jnp.min(jnp.where(s == m, gcol, INT_MAX), axis=1, keepdims=True)
        s = jnp.where(gcol == sel0, NEG_INF, s)
        m1 = jnp.max(s, axis=1, keepdims=True)
        sel1 = jnp.min(jnp.where(s == m1, gcol, INT_MAX), axis=1, keepdims=True)
        s = jnp.where(gcol == sel1, NEG_INF, s)
        sim_ref[...] = s
        la = 16 + 2 * t
        cv = jnp.where(lanec == la, m, cv_ref[...])
        cv_ref[...] = jnp.where(lanec == la + 1, m1, cv)
        ci = jnp.where(lanec == la, sel0, ci_ref[...])
        ci_ref[...] = jnp.where(lanec == la + 1, sel1, ci)
        return t + 1, jnp.max(s, axis=1, keepdims=True)

    m0 = jnp.max(sim_ref[...], axis=1, keepdims=True)
    tf, _ = lax.while_loop(pass_cond, pass_body, (jnp.int32(0), m0))
    contributed = tf > 0

    @pl.when(contributed)
    def _merge():
        nv_ref[...] = jnp.full((Q_BLK, 16), NEG_INF, jnp.float32)
        ni_ref[...] = jnp.full((Q_BLK, 16), INT_MAX, jnp.int32)

        def merge_body(t, carry):
            cv = cv_ref[...]
            ci = ci_ref[...]
            m = jnp.max(cv, axis=1, keepdims=True)
            sel = jnp.min(jnp.where(cv == m, ci, INT_MAX), axis=1,
                          keepdims=True)
            cv_ref[...] = jnp.where(ci == sel, NEG_INF, cv)
            nv_ref[...] = jnp.where(lane16 == t, m, nv_ref[...])
            ni_ref[...] = jnp.where(lane16 == t, sel, ni_ref[...])
            return carry

        lax.fori_loop(0, TOP_K, merge_body, 0)

        cv_ref[...] = jnp.concatenate(
            [nv_ref[...], jnp.full((Q_BLK, 16), NEG_INF, jnp.float32)], axis=1)
        ci_ref[...] = jnp.concatenate(
            [ni_ref[...], jnp.full((Q_BLK, 16), INT_MAX, jnp.int32)], axis=1)

    @pl.when(k == NKB - 1)
    def _emit():
        g = ni_ref[...]
        out_ref[...] = jnp.where(lane16 < TOP_K, g,
                                 jnp.broadcast_to(g[:, :1], (Q_BLK, 16)))


def _topk_call(y_pred, feats_pad, interpret=False):
    return pl.pallas_call(
        _topk_body,
        grid=(NQB, NKB),
        in_specs=[
            pl.BlockSpec((Q_BLK, D), lambda q, k: (q, 0)),
            pl.BlockSpec((K_BLK, D), lambda q, k: (k, 0)),
        ],
        out_specs=pl.BlockSpec((Q_BLK, 16), lambda q, k: (q, 0)),
        out_shape=jax.ShapeDtypeStruct((Q, 16), jnp.int32),
        scratch_shapes=[
            pltpu.VMEM((Q_BLK, K_BLK), jnp.float32),
            pltpu.VMEM((Q_BLK, 32), jnp.float32),
            pltpu.VMEM((Q_BLK, 32), jnp.int32),
            pltpu.VMEM((Q_BLK, 16), jnp.float32),
            pltpu.VMEM((Q_BLK, 16), jnp.int32),
        ],
        compiler_params=pltpu.CompilerParams(
            dimension_semantics=("arbitrary", "arbitrary"),
        ),
        interpret=interpret,
    )(y_pred, feats_pad)


def _make_scatter_call():
    info = plsc.get_sparse_core_info()
    nc, ns = info.num_cores, info.num_subcores
    nw = nc * ns
    qpw = Q // nw
    mesh = plsc.VectorSubcoreMesh(core_axis_name="c", subcore_axis_name="s")

    @functools.partial(
        pl.kernel,
        mesh=mesh,
        out_type=jax.ShapeDtypeStruct((Q, OUT_PAD), jnp.float32),
        scratch_types=[
            pltpu.VMEM((K_DB,), jnp.int32),
            pltpu.VMEM((qpw, 16), jnp.int32),
            pltpu.VMEM((OUT_PAD,), jnp.float32),
        ],
        compiler_params=pltpu.CompilerParams(needs_layout_passes=False),
    )
    def scatter_kernel(idx_hbm, y_hbm, out_hbm, ytile, idxv, row):
        wid = lax.axis_index("s") * nc + lax.axis_index("c")
        base = wid * qpw
        pltpu.sync_copy(y_hbm, ytile)
        pltpu.sync_copy(idx_hbm.at[pl.ds(base, qpw)], idxv)

        zeros16 = jnp.zeros((16,), jnp.float32)
        ones16 = jnp.ones((16,), jnp.float32)

        def zero_body(i, carry):
            row[pl.ds(i * 16, 16)] = zeros16
            return carry

        lax.fori_loop(0, OUT_PAD // 16, zero_body, 0)

        def q_body(q, carry):
            idx16 = idxv[q, :]
            labels = plsc.load_gather(ytile, [idx16])
            plsc.store_scatter(row, [labels], ones16)
            pltpu.sync_copy(row, out_hbm.at[base + q])
            plsc.store_scatter(row, [labels], zeros16)
            return carry

        lax.fori_loop(0, qpw, q_body, 0)

    return scatter_kernel


def kernel(y_pred, image_features, y):
    y_pred = y_pred / jnp.linalg.norm(y_pred, axis=-1, keepdims=True)
    feats_pad = jnp.zeros((K_PAD, D), jnp.float32).at[:K_DB].set(image_features)
    idx16 = _topk_call(y_pred, feats_pad)
    out_pad = _make_scatter_call()(idx16, y)
    return out_pad[:, :NUM_CLASSES]


# speedup vs baseline: 1.0102x; 1.0102x over previous
import functools

import jax
import jax.numpy as jnp
from jax import lax
from jax.experimental import pallas as pl
from jax.experimental.pallas import tpu as pltpu
from jax.experimental.pallas import tpu_sc as plsc

Q = 1024
D = 64
K_DB = 100000
NUM_CLASSES = 1000
TOP_K = 10

K_BLK = 2048
NKB = -(-K_DB // K_BLK)
K_PAD = K_BLK * NKB

OUT_PAD = 1024
INT_MAX = 2**31 - 1
NEG_INF = float("-inf")


Q_BLK = 1024
NQB = Q // Q_BLK
CAND_W = 32


def _topk_body(ypred_ref, feats_ref, out_ref,
               sim_ref, cv_ref, ci_ref, nv_ref, ni_ref):
    k = pl.program_id(1)

    @pl.when(k == 0)
    def _init():
        cv_ref[...] = jnp.full((Q_BLK, CAND_W), NEG_INF, jnp.float32)
        ci_ref[...] = jnp.full((Q_BLK, CAND_W), INT_MAX, jnp.int32)

    gcol = lax.broadcasted_iota(jnp.int32, (Q_BLK, K_BLK), 1) + k * K_BLK
    sim = lax.dot_general(
        ypred_ref[...], feats_ref[...],
        (((1,), (1,)), ((), ())),
        preferred_element_type=jnp.float32,
    )

    @pl.when(k == NKB - 1)
    def _store_masked():
        sim_ref[...] = jnp.where(gcol < K_DB, sim, NEG_INF)

    @pl.when(k < NKB - 1)
    def _store_plain():
        sim_ref[...] = sim

    lanec = lax.broadcasted_iota(jnp.int32, (Q_BLK, CAND_W), 1)
    lane16 = lax.broadcasted_iota(jnp.int32, (Q_BLK, 16), 1)

    thresh = cv_ref[:, 9:10]

    def pass_cond(state):
        t, m = state
        return jnp.logical_and(t < TOP_K, jnp.any(m > thresh))

    def pass_body(state):
        t, m = state
        s = sim_ref[...]
        sel = jnp.min(jnp.where(s == m, gcol, INT_MAX), axis=1, keepdims=True)
        s = jnp.where(gcol == sel, NEG_INF, s)
        sim_ref[...] = s
        cv_ref[...] = jnp.where(lanec == 16 + t, m, cv_ref[...])
        ci_ref[...] = jnp.where(lanec == 16 + t, sel, ci_ref[...])
        return t + 1, jnp.max(s, axis=1, keepdims=True)

    m0 = jnp.max(sim_ref[...], axis=1, keepdims=True)
    tf, _ = lax.while_loop(pass_cond, pass_body, (jnp.int32(0), m0))
    contributed = tf > 0

    @pl.when(contributed)
    def _merge():
        nv_ref[...] = jnp.full((Q_BLK, 16), NEG_INF, jnp.float32)
        ni_ref[...] = jnp.full((Q_BLK, 16), INT_MAX, jnp.int32)

        def merge_body(t, carry):
            cv = cv_ref[...]
            ci = ci_ref[...]
            m = jnp.max(cv, axis=1, keepdims=True)
            sel = jnp.min(jnp.where(cv == m, ci, INT_MAX), axis=1,
                          keepdims=True)
            cv_ref[...] = jnp.where(ci == sel, NEG_INF, cv)
            nv_ref[...] = jnp.where(lane16 == t, m, nv_ref[...])
            ni_ref[...] = jnp.where(lane16 == t, sel, ni_ref[...])
            return carry

        lax.fori_loop(0, TOP_K, merge_body, 0)

        cv_ref[...] = jnp.concatenate(
            [nv_ref[...], jnp.full((Q_BLK, 16), NEG_INF, jnp.float32)], axis=1)
        ci_ref[...] = jnp.concatenate(
            [ni_ref[...], jnp.full((Q_BLK, 16), INT_MAX, jnp.int32)], axis=1)

    @pl.when(k == NKB - 1)
    def _emit():
        g = ni_ref[...]
        out_ref[...] = jnp.where(lane16 < TOP_K, g,
                                 jnp.broadcast_to(g[:, :1], (Q_BLK, 16)))


def _topk_call(y_pred, feats_pad, interpret=False):
    return pl.pallas_call(
        _topk_body,
        grid=(NQB, NKB),
        in_specs=[
            pl.BlockSpec((Q_BLK, D), lambda q, k: (q, 0)),
            pl.BlockSpec((K_BLK, D), lambda q, k: (k, 0)),
        ],
        out_specs=pl.BlockSpec((Q_BLK, 16), lambda q, k: (q, 0)),
        out_shape=jax.ShapeDtypeStruct((Q, 16), jnp.int32),
        scratch_shapes=[
            pltpu.VMEM((Q_BLK, K_BLK), jnp.float32),
            pltpu.VMEM((Q_BLK, 32), jnp.float32),
            pltpu.VMEM((Q_BLK, 32), jnp.int32),
            pltpu.VMEM((Q_BLK, 16), jnp.float32),
            pltpu.VMEM((Q_BLK, 16), jnp.int32),
        ],
        compiler_params=pltpu.CompilerParams(
            dimension_semantics=("arbitrary", "arbitrary"),
        ),
        interpret=interpret,
    )(y_pred, feats_pad)


def _make_scatter_call():
    info = plsc.get_sparse_core_info()
    nc, ns = info.num_cores, info.num_subcores
    nw = nc * ns
    qpw = Q // nw
    mesh = plsc.VectorSubcoreMesh(core_axis_name="c", subcore_axis_name="s")

    @functools.partial(
        pl.kernel,
        mesh=mesh,
        out_type=jax.ShapeDtypeStruct((Q, OUT_PAD), jnp.float32),
        scratch_types=[
            pltpu.VMEM((K_DB,), jnp.int32),
            pltpu.VMEM((qpw, 16), jnp.int32),
            pltpu.VMEM((OUT_PAD,), jnp.float32),
        ],
        compiler_params=pltpu.CompilerParams(needs_layout_passes=False),
    )
    def scatter_kernel(idx_hbm, y_hbm, out_hbm, ytile, idxv, row):
        wid = lax.axis_index("s") * nc + lax.axis_index("c")
        base = wid * qpw
        pltpu.sync_copy(y_hbm, ytile)
        pltpu.sync_copy(idx_hbm.at[pl.ds(base, qpw)], idxv)

        zeros16 = jnp.zeros((16,), jnp.float32)
        ones16 = jnp.ones((16,), jnp.float32)

        def zero_body(i, carry):
            row[pl.ds(i * 16, 16)] = zeros16
            return carry

        lax.fori_loop(0, OUT_PAD // 16, zero_body, 0)

        def q_body(q, carry):
            idx16 = idxv[q, :]
            labels = plsc.load_gather(ytile, [idx16])
            plsc.store_scatter(row, [labels], ones16)
            pltpu.sync_copy(row, out_hbm.at[base + q])
            plsc.store_scatter(row, [labels], zeros16)
            return carry

        lax.fori_loop(0, qpw, q_body, 0)

    return scatter_kernel


def kernel(y_pred, image_features, y):
    y_pred = y_pred / jnp.linalg.norm(y_pred, axis=-1, keepdims=True)
    feats_pad = jnp.zeros((K_PAD, D), jnp.float32).at[:K_DB].set(image_features)
    idx16 = _topk_call(y_pred, feats_pad)
    out_pad = _make_scatter_call()(idx16, y)
    return out_pad[:, :NUM_CLASSES]
